# baseline (speedup 1.0000x reference)
import functools

import jax
import jax.numpy as jnp
from jax import lax
from jax.experimental import pallas as pl
from jax.experimental.pallas import tpu as pltpu
from jax.experimental.pallas import tpu_sc as plsc

CHUNK = 128
NBUF = 4
IBLK = 28
N_DUMP = 48
ZROWS = 1564
RB = 2000

_SC_PARAMS = pltpu.CompilerParams(use_tc_tiling_on_sc=False)


def _sc_pass(d, nchunk, np_rows, with_gather):
    mesh = plsc.VectorSubcoreMesh(core_axis_name="c", subcore_axis_name="s")
    rpt = np_rows // 16
    nblk = nchunk // IBLK
    nsb = IBLK // NBUF

    scratch = [
        pltpu.VMEM((IBLK, CHUNK), jnp.int32),
        pltpu.VMEM((IBLK, CHUNK), jnp.int32),
        pltpu.VMEM((NBUF, CHUNK, d), jnp.float32),
        pltpu.VMEM_SHARED((np_rows, d), jnp.float32),
        pltpu.SemaphoreType.DMA((NBUF,)),
        pltpu.SemaphoreType.DMA((NBUF,)),
    ]

    @functools.partial(
        pl.kernel, mesh=mesh,
        out_type=jax.ShapeDtypeStruct((2, np_rows, d), jnp.float32),
        scratch_types=scratch, compiler_params=_SC_PARAMS)
    def scat(s_hbm, srcp_hbm, dstp_hbm, zeros_hbm, out_hbm,
             src_v, dst_v, rows_v, acc, gsem, ssem):
        c = lax.axis_index("c")
        s = lax.axis_index("s")
        tid = c * 16 + s

        for z in range(rpt // ZROWS):
            pltpu.sync_copy(zeros_hbm, acc.at[pl.ds(s * rpt + z * ZROWS,
                                                    ZROWS)])
        if not with_gather:
            pltpu.sync_copy(s_hbm, rows_v.at[0])
        plsc.subcore_barrier()

        if with_gather:
            def blk(ob, carry):
                pltpu.sync_copy(srcp_hbm.at[tid, ob], src_v)
                pltpu.sync_copy(dstp_hbm.at[tid, ob], dst_v)

                def sb(i, carry2):
                    base = i * NBUF
                    first = (ob == 0) & (i == 0)
                    for k in range(NBUF):
                        @pl.when(jnp.logical_not(first))
                        def _():
                            pltpu.make_async_copy(
                                rows_v.at[k], acc.at[dst_v.at[base + k]],
                                ssem.at[k]).wait()
                        pltpu.async_copy(s_hbm.at[src_v.at[base + k]],
                                         rows_v.at[k], gsem.at[k])
                    for k in range(NBUF):
                        pltpu.make_async_copy(s_hbm.at[src_v.at[base + k]],
                                              rows_v.at[k],
                                              gsem.at[k]).wait()
                        pltpu.async_copy(rows_v.at[k],
                                         acc.at[dst_v.at[base + k]],
                                         ssem.at[k], add=True)
                    return carry2

                return lax.fori_loop(0, nsb, sb, carry)

            lax.fori_loop(0, nblk, blk, 0)
            for k in range(NBUF):
                pltpu.make_async_copy(rows_v.at[k], acc.at[dst_v.at[k]],
                                      ssem.at[k]).wait()
        else:
            def blk(ob, carry):
                pltpu.sync_copy(dstp_hbm.at[tid, ob], dst_v)

                def sb(i, carry2):
                    base = i * NBUF
                    first = (ob == 0) & (i == 0)
                    for k in range(NBUF):
                        @pl.when(jnp.logical_not(first))
                        def _():
                            pltpu.make_async_copy(
                                rows_v.at[0], acc.at[dst_v.at[base + k]],
                                ssem.at[k]).wait()
                        pltpu.async_copy(rows_v.at[0],
                                         acc.at[dst_v.at[base + k]],
                                         ssem.at[k], add=True)
                    return carry2

                return lax.fori_loop(0, nsb, sb, carry)

            lax.fori_loop(0, nblk, blk, 0)
            for k in range(NBUF):
                pltpu.make_async_copy(rows_v.at[0], acc.at[dst_v.at[k]],
                                      ssem.at[k]).wait()

        plsc.subcore_barrier()
        pltpu.sync_copy(acc.at[pl.ds(s * rpt, rpt)],
                        out_hbm.at[c, pl.ds(s * rpt, rpt)])

    return scat


def _sc_pass32(nchunk, np_rows):
    mesh = plsc.VectorSubcoreMesh(core_axis_name="c", subcore_axis_name="s")
    rpt = np_rows // 16
    nblk = nchunk // IBLK
    nsb = IBLK // NBUF

    scratch = [
        pltpu.VMEM((IBLK, CHUNK), jnp.int32),
        pltpu.VMEM((IBLK, CHUNK), jnp.int32),
        pltpu.VMEM((NBUF, CHUNK, 16), jnp.float32),
        pltpu.VMEM_SHARED((np_rows, 16), jnp.float32),
        pltpu.SemaphoreType.DMA((NBUF,)),
        pltpu.SemaphoreType.DMA((NBUF,)),
    ]

    @functools.partial(
        pl.kernel, mesh=mesh,
        out_type=jax.ShapeDtypeStruct((np_rows, 32), jnp.float32),
        scratch_types=scratch, compiler_params=_SC_PARAMS)
    def scat(sst_hbm, srcp_hbm, dstp_hbm, zeros_hbm, out_hbm,
             src_v, dst_v, rows_v, acc, gsem, ssem):
        c = lax.axis_index("c")
        s = lax.axis_index("s")

        for z in range(rpt // ZROWS):
            pltpu.sync_copy(zeros_hbm, acc.at[pl.ds(s * rpt + z * ZROWS,
                                                    ZROWS)])
        plsc.subcore_barrier()

        def run(s_tab):
            def blk(ob, carry):
                row = 2 * s + ob // nblk
                obb = lax.rem(ob, nblk)
                pltpu.sync_copy(srcp_hbm.at[row, obb], src_v)
                pltpu.sync_copy(dstp_hbm.at[row, obb], dst_v)

                def sb(i, carry2):
                    base = i * NBUF
                    first = (ob == 0) & (i == 0)
                    for k in range(NBUF):
                        @pl.when(jnp.logical_not(first))
                        def _():
                            pltpu.make_async_copy(
                                rows_v.at[k], acc.at[dst_v.at[base + k]],
                                ssem.at[k]).wait()
                        pltpu.async_copy(s_tab.at[src_v.at[base + k]],
                                         rows_v.at[k], gsem.at[k])
                    for k in range(NBUF):
                        pltpu.make_async_copy(s_tab.at[src_v.at[base + k]],
                                              rows_v.at[k],
                                              gsem.at[k]).wait()
                        pltpu.async_copy(rows_v.at[k],
                                         acc.at[dst_v.at[base + k]],
                                         ssem.at[k], add=True)
                    return carry2

                return lax.fori_loop(0, nsb, sb, carry)

            lax.fori_loop(0, 2 * nblk, blk, 0)
            for k in range(NBUF):
                pltpu.make_async_copy(rows_v.at[k], acc.at[dst_v.at[k]],
                                      ssem.at[k]).wait()

        @pl.when(c == 0)
        def _():
            run(sst_hbm.at[0])

        @pl.when(c == 1)
        def _():
            run(sst_hbm.at[1])

        plsc.subcore_barrier()
        pltpu.sync_copy(acc.at[pl.ds(s * rpt, rpt)],
                        out_hbm.at[pl.ds(s * rpt, rpt), pl.ds(c * 16, 16)])

    return scat


def _row_spec(d):
    return pl.BlockSpec((RB, d), lambda i: (i, 0))


def _part_spec(d):
    return pl.BlockSpec((2, RB, d), lambda i: (0, i, 0))


def _full_spec(r, ncol):
    return pl.BlockSpec((r, ncol), lambda i: (0, 0))


def _tc_call(body, in_specs, out_specs, out_shape, grid):
    return pl.pallas_call(body, grid=grid, in_specs=in_specs,
                          out_specs=out_specs, out_shape=out_shape)


def _tc1_body(t_ref, coords_ref, atf_ref, e_ref, dinv_ref, sp1_ref):
    t = t_ref[...]
    deg = t[0, :, 0:1] + t[1, :, 0:1] + 1.0
    dinv = lax.rsqrt(deg)
    at = atf_ref[...]
    e = e_ref[...]
    emb = jnp.where(at < 0.5, e[0:1, 0:3], e[1:2, 0:3])
    feats = jnp.concatenate([coords_ref[...], emb], axis=1)
    sp = feats * dinv
    pad = jnp.zeros((sp.shape[0], 10), jnp.float32)
    sp1_ref[...] = jnp.concatenate([sp, pad], axis=1)
    dinv_ref[...] = dinv


def _stack16(x):
    return jnp.concatenate([x[None, :, 0:16], x[None, :, 16:32]], axis=0)


def _tc2_body(t_ref, sp_ref, dinv_ref, w_ref, b_ref, out_ref):
    t = t_ref[...]
    dinv = dinv_ref[...]
    u = dinv * (t[0] + t[1] + sp_ref[...])
    x = jnp.maximum(jnp.dot(u, w_ref[...],
                            preferred_element_type=jnp.float32) + b_ref[...],
                    0.0)
    out_ref[...] = _stack16(x * dinv)


def _tc3_body(t_ref, sp_ref, dinv_ref, w_ref, b_ref, oa_ref, ob_ref):
    dinv = dinv_ref[...]
    sp = sp_ref[...]
    spc = jnp.concatenate([sp[0], sp[1]], axis=1)
    u = dinv * (t_ref[...] + spc)
    x = jnp.maximum(jnp.dot(u, w_ref[...],
                            preferred_element_type=jnp.float32) + b_ref[...],
                    0.0)
    sp3 = x * dinv
    oa_ref[...] = _stack16(sp3[:, 0:32])
    ob_ref[...] = _stack16(sp3[:, 32:64])


def _tc4_body(ta_ref, tb_ref, spa_ref, spb_ref, dinv_ref, w3_ref, b3_ref,
              w4_ref, out_ref):
    dinv = dinv_ref[...]
    spa = spa_ref[...]
    spb = spb_ref[...]
    sp = jnp.concatenate([spa[0], spa[1], spb[0], spb[1]], axis=1)
    t = jnp.concatenate([ta_ref[...], tb_ref[...]], axis=1)
    u = dinv * (t + sp)
    x4 = jnp.maximum(jnp.dot(u, w3_ref[...],
                             preferred_element_type=jnp.float32) + b3_ref[...],
                     0.0)
    h4 = jnp.dot(x4, w4_ref[...], preferred_element_type=jnp.float32)
    out_ref[...] = _stack16(h4 * dinv)


def _tc5_body(t_ref, sp_ref, dinv_ref, b4_ref, w5_ref, out_ref):
    dinv = dinv_ref[...]
    sp = sp_ref[...]
    spc = jnp.concatenate([sp[0], sp[1]], axis=1)
    x5 = jnp.maximum(dinv * (t_ref[...] + spc) + b4_ref[...], 0.0)
    h5 = jnp.dot(x5, w5_ref[...], preferred_element_type=jnp.float32)
    out_ref[...] = h5 * dinv


def _tc6_body(t_ref, sp_ref, dinv_ref, b5_ref, out_ref):
    t = t_ref[...]
    dinv = dinv_ref[...]
    y = dinv * (t[0] + t[1] + sp_ref[...]) + b5_ref[...]
    out_ref[...] = y[:, 0:3]


def kernel(noisy_coords, atom_types, noisy_edge_index, atom_emb,
           W1, b1, W2, b2, W3, b3, W4, b4, W5, b5):
    n = noisy_coords.shape[0]
    e = noisy_edge_index.shape[1]
    np_rows = n + N_DUMP

    nchunk = -(-e // (32 * CHUNK))
    nchunk += (-nchunk) % IBLK
    ep = 32 * nchunk * CHUNK
    padlen = ep - e
    src = noisy_edge_index[0]
    dst = noisy_edge_index[1]
    pad_i = jnp.arange(padlen, dtype=jnp.int32)
    src_p = jnp.concatenate([src, (pad_i * 9973) % n])
    dst_p = jnp.concatenate([dst, n + (pad_i % 8)])
    srcp = src_p.reshape(32, nchunk // IBLK, IBLK, CHUNK)
    dstp = dst_p.reshape(32, nchunk // IBLK, IBLK, CHUNK)

    zeros16 = jnp.zeros((ZROWS, 16), jnp.float32)
    ones16 = jnp.ones((CHUNK, 16), jnp.float32)

    deg_pass = _sc_pass(16, nchunk, np_rows, with_gather=False)
    scat16 = _sc_pass(16, nchunk, np_rows, with_gather=True)
    scat32 = _sc_pass32(nchunk, np_rows)

    w1p = jnp.concatenate([W1, jnp.zeros((10, 32), jnp.float32)], axis=0)
    w5p = jnp.concatenate([W5, jnp.zeros((32, 13), jnp.float32)], axis=1)
    b1r = b1.reshape(1, -1)
    b2r = b2.reshape(1, -1)
    b3r = b3.reshape(1, -1)
    b4r = b4.reshape(1, -1)
    b5r = jnp.concatenate([b5, jnp.zeros((13,), jnp.float32)]).reshape(1, -1)
    e_pad = jnp.zeros((8, 128), jnp.float32).at[0:2, 0:3].set(atom_emb)
    atf = atom_types.astype(jnp.float32).reshape(n, 1)

    grid = (n // RB,)

    t_deg = deg_pass(ones16, srcp, dstp, zeros16)

    tc1 = _tc_call(
        _tc1_body,
        [_part_spec(16), _row_spec(3), _row_spec(1), _full_spec(8, 128)],
        [_row_spec(1), _row_spec(16)],
        (jax.ShapeDtypeStruct((n, 1), jnp.float32),
         jax.ShapeDtypeStruct((n, 16), jnp.float32)),
        grid)
    dinv, sp1 = tc1(t_deg, noisy_coords, atf, e_pad)

    t1 = scat16(sp1, srcp, dstp, zeros16)
    tc2 = _tc_call(
        _tc2_body,
        [_part_spec(16), _row_spec(16), _row_spec(1), _full_spec(16, 32),
         _full_spec(1, 32)],
        _part_spec(16),
        jax.ShapeDtypeStruct((2, n, 16), jnp.float32),
        grid)
    sp2 = tc2(t1, sp1, dinv, w1p, b1r)

    t2 = scat32(sp2, srcp, dstp, zeros16)
    tc3 = _tc_call(
        _tc3_body,
        [_row_spec(32), _part_spec(16), _row_spec(1), _full_spec(32, 64),
         _full_spec(1, 64)],
        [_part_spec(16), _part_spec(16)],
        (jax.ShapeDtypeStruct((2, n, 16), jnp.float32),
         jax.ShapeDtypeStruct((2, n, 16), jnp.float32)),
        grid)
    sp3a, sp3b = tc3(t2, sp2, dinv, W2, b2r)

    t3a = scat32(sp3a, srcp, dstp, zeros16)
    t3b = scat32(sp3b, srcp, dstp, zeros16)
    tc4 = _tc_call(
        _tc4_body,
        [_row_spec(32), _row_spec(32), _part_spec(16), _part_spec(16),
         _row_spec(1), _full_spec(64, 64), _full_spec(1, 64),
         _full_spec(64, 32)],
        _part_spec(16),
        jax.ShapeDtypeStruct((2, n, 16), jnp.float32),
        grid)
    sp4 = tc4(t3a, t3b, sp3a, sp3b, dinv, W3, b3r, W4)

    t4 = scat32(sp4, srcp, dstp, zeros16)
    tc5 = _tc_call(
        _tc5_body,
        [_row_spec(32), _part_spec(16), _row_spec(1), _full_spec(1, 32),
         _full_spec(32, 16)],
        _row_spec(16),
        jax.ShapeDtypeStruct((n, 16), jnp.float32),
        grid)
    sp5 = tc5(t4, sp4, dinv, b4r, w5p)

    t5 = scat16(sp5, srcp, dstp, zeros16)
    tc6 = _tc_call(
        _tc6_body,
        [_part_spec(16), _row_spec(16), _row_spec(1), _full_spec(1, 16)],
        _row_spec(3),
        jax.ShapeDtypeStruct((n, 3), jnp.float32),
        grid)
    return tc6(t5, sp5, dinv, b5r)


# speedup vs baseline: 1.0388x; 1.0388x over previous
import functools

import jax
import jax.numpy as jnp
from jax import lax
from jax.experimental import pallas as pl
from jax.experimental.pallas import tpu as pltpu
from jax.experimental.pallas import tpu_sc as plsc

CHUNK = 128
NBUF = 4
IBLK = 28
N_DUMP = 48
ZROWS = 1564
RB = 2000

_SC_PARAMS = pltpu.CompilerParams(use_tc_tiling_on_sc=False)


def _sc_pass(d, nchunk, np_rows, with_gather):
    mesh = plsc.VectorSubcoreMesh(core_axis_name="c", subcore_axis_name="s")
    rpt = np_rows // 16
    nblk = nchunk // IBLK
    nsb = IBLK // NBUF

    scratch = [
        pltpu.VMEM((IBLK, CHUNK), jnp.int32),
        pltpu.VMEM((IBLK, CHUNK), jnp.int32),
        pltpu.VMEM((NBUF, CHUNK, d), jnp.float32),
        pltpu.VMEM_SHARED((np_rows, d), jnp.float32),
        pltpu.SemaphoreType.DMA((NBUF,)),
        pltpu.SemaphoreType.DMA((NBUF,)),
    ]

    @functools.partial(
        pl.kernel, mesh=mesh,
        out_type=jax.ShapeDtypeStruct((2, np_rows, d), jnp.float32),
        scratch_types=scratch, compiler_params=_SC_PARAMS)
    def scat(s_hbm, srcp_hbm, dstp_hbm, zeros_hbm, out_hbm,
             src_v, dst_v, rows_v, acc, gsem, ssem):
        c = lax.axis_index("c")
        s = lax.axis_index("s")
        tid = c * 16 + s

        for z in range(rpt // ZROWS):
            pltpu.sync_copy(zeros_hbm, acc.at[pl.ds(s * rpt + z * ZROWS,
                                                    ZROWS)])
        if not with_gather:
            pltpu.sync_copy(s_hbm, rows_v.at[0])
        plsc.subcore_barrier()

        if with_gather:
            def blk(ob, carry):
                pltpu.sync_copy(srcp_hbm.at[tid, ob], src_v)
                pltpu.sync_copy(dstp_hbm.at[tid, ob], dst_v)

                def sb(i, carry2):
                    base = i * NBUF
                    first = (ob == 0) & (i == 0)
                    for k in range(NBUF):
                        @pl.when(jnp.logical_not(first))
                        def _():
                            pltpu.make_async_copy(
                                rows_v.at[k], acc.at[dst_v.at[base + k]],
                                ssem.at[k]).wait()
                        pltpu.async_copy(s_hbm.at[src_v.at[base + k]],
                                         rows_v.at[k], gsem.at[k])
                    for k in range(NBUF):
                        pltpu.make_async_copy(s_hbm.at[src_v.at[base + k]],
                                              rows_v.at[k],
                                              gsem.at[k]).wait()
                        pltpu.async_copy(rows_v.at[k],
                                         acc.at[dst_v.at[base + k]],
                                         ssem.at[k], add=True)
                    return carry2

                return lax.fori_loop(0, nsb, sb, carry)

            lax.fori_loop(0, nblk, blk, 0)
            for k in range(NBUF):
                pltpu.make_async_copy(rows_v.at[k], acc.at[dst_v.at[k]],
                                      ssem.at[k]).wait()
        else:
            def blk(ob, carry):
                pltpu.sync_copy(dstp_hbm.at[tid, ob], dst_v)

                def sb(i, carry2):
                    base = i * NBUF
                    first = (ob == 0) & (i == 0)
                    for k in range(NBUF):
                        @pl.when(jnp.logical_not(first))
                        def _():
                            pltpu.make_async_copy(
                                rows_v.at[0], acc.at[dst_v.at[base + k]],
                                ssem.at[k]).wait()
                        pltpu.async_copy(rows_v.at[0],
                                         acc.at[dst_v.at[base + k]],
                                         ssem.at[k], add=True)
                    return carry2

                return lax.fori_loop(0, nsb, sb, carry)

            lax.fori_loop(0, nblk, blk, 0)
            for k in range(NBUF):
                pltpu.make_async_copy(rows_v.at[0], acc.at[dst_v.at[k]],
                                      ssem.at[k]).wait()

        plsc.subcore_barrier()
        pltpu.sync_copy(acc.at[pl.ds(s * rpt, rpt)],
                        out_hbm.at[c, pl.ds(s * rpt, rpt)])

    return scat


def _sc_pass32(nchunk, np_rows):
    mesh = plsc.VectorSubcoreMesh(core_axis_name="c", subcore_axis_name="s")
    rpt = np_rows // 16
    nblk = nchunk // IBLK
    nsb = IBLK // NBUF

    scratch = [
        pltpu.VMEM((IBLK, CHUNK), jnp.int32),
        pltpu.VMEM((IBLK, CHUNK), jnp.int32),
        pltpu.VMEM((NBUF, CHUNK, 16), jnp.float32),
        pltpu.VMEM_SHARED((np_rows, 16), jnp.float32),
        pltpu.VMEM_SHARED((np_rows, 16), jnp.float32),
        pltpu.SemaphoreType.DMA((NBUF,)),
        pltpu.SemaphoreType.DMA((NBUF,)),
    ]
    tail = 0

    @functools.partial(
        pl.kernel, mesh=mesh,
        out_type=jax.ShapeDtypeStruct((np_rows, 32), jnp.float32),
        scratch_types=scratch, compiler_params=_SC_PARAMS)
    def scat(sst_hbm, srcp_hbm, dstp_hbm, zeros_hbm, out_hbm,
             src_v, dst_v, rows_v, acc, s_sh, gsem, ssem):
        c = lax.axis_index("c")
        s = lax.axis_index("s")
        nrow = sst_hbm.shape[1]
        last = nrow - 15 * rpt

        for z in range(rpt // ZROWS):
            pltpu.sync_copy(zeros_hbm, acc.at[pl.ds(s * rpt + z * ZROWS,
                                                    ZROWS)])

        @pl.when(s < 15)
        def _():
            pltpu.sync_copy(sst_hbm.at[c, pl.ds(s * rpt, rpt)],
                            s_sh.at[pl.ds(s * rpt, rpt)])

        @pl.when(s == 15)
        def _():
            pltpu.sync_copy(sst_hbm.at[c, pl.ds(15 * rpt, last)],
                            s_sh.at[pl.ds(15 * rpt, last)])

        plsc.subcore_barrier()

        def run(s_tab):
            def blk(ob, carry):
                row = 2 * s + ob // nblk
                obb = lax.rem(ob, nblk)
                pltpu.sync_copy(srcp_hbm.at[row, obb], src_v)
                pltpu.sync_copy(dstp_hbm.at[row, obb], dst_v)

                def sb(i, carry2):
                    base = i * NBUF
                    first = (ob == 0) & (i == 0)
                    for k in range(NBUF):
                        @pl.when(jnp.logical_not(first))
                        def _():
                            pltpu.make_async_copy(
                                rows_v.at[k], acc.at[dst_v.at[base + k]],
                                ssem.at[k]).wait()
                        pltpu.async_copy(s_tab.at[src_v.at[base + k]],
                                         rows_v.at[k], gsem.at[k])
                    for k in range(NBUF):
                        pltpu.make_async_copy(s_tab.at[src_v.at[base + k]],
                                              rows_v.at[k],
                                              gsem.at[k]).wait()
                        pltpu.async_copy(rows_v.at[k],
                                         acc.at[dst_v.at[base + k]],
                                         ssem.at[k], add=True)
                    return carry2

                return lax.fori_loop(0, nsb, sb, carry)

            lax.fori_loop(0, 2 * nblk, blk, 0)
            for k in range(NBUF):
                pltpu.make_async_copy(rows_v.at[k], acc.at[dst_v.at[k]],
                                      ssem.at[k]).wait()

        run(s_sh)

        plsc.subcore_barrier()
        pltpu.sync_copy(acc.at[pl.ds(s * rpt, rpt)],
                        out_hbm.at[pl.ds(s * rpt, rpt), pl.ds(c * 16, 16)])

    return scat


def _row_spec(d):
    return pl.BlockSpec((RB, d), lambda i: (i, 0))


def _part_spec(d):
    return pl.BlockSpec((2, RB, d), lambda i: (0, i, 0))


def _full_spec(r, ncol):
    return pl.BlockSpec((r, ncol), lambda i: (0, 0))


def _tc_call(body, in_specs, out_specs, out_shape, grid):
    return pl.pallas_call(body, grid=grid, in_specs=in_specs,
                          out_specs=out_specs, out_shape=out_shape)


def _tc1_body(t_ref, coords_ref, atf_ref, e_ref, dinv_ref, sp1_ref):
    t = t_ref[...]
    deg = t[0, :, 0:1] + t[1, :, 0:1] + 1.0
    dinv = lax.rsqrt(deg)
    at = atf_ref[...]
    e = e_ref[...]
    emb = jnp.where(at < 0.5, e[0:1, 0:3], e[1:2, 0:3])
    feats = jnp.concatenate([coords_ref[...], emb], axis=1)
    sp = feats * dinv
    pad = jnp.zeros((sp.shape[0], 10), jnp.float32)
    sp1_ref[...] = jnp.concatenate([sp, pad], axis=1)
    dinv_ref[...] = dinv


def _stack16(x):
    return jnp.concatenate([x[None, :, 0:16], x[None, :, 16:32]], axis=0)


def _tc2_body(t_ref, sp_ref, dinv_ref, w_ref, b_ref, out_ref):
    t = t_ref[...]
    dinv = dinv_ref[...]
    u = dinv * (t[0] + t[1] + sp_ref[...])
    x = jnp.maximum(jnp.dot(u, w_ref[...],
                            preferred_element_type=jnp.float32) + b_ref[...],
                    0.0)
    out_ref[...] = _stack16(x * dinv)


def _tc3_body(t_ref, sp_ref, dinv_ref, w_ref, b_ref, oa_ref, ob_ref):
    dinv = dinv_ref[...]
    sp = sp_ref[...]
    spc = jnp.concatenate([sp[0], sp[1]], axis=1)
    u = dinv * (t_ref[...] + spc)
    x = jnp.maximum(jnp.dot(u, w_ref[...],
                            preferred_element_type=jnp.float32) + b_ref[...],
                    0.0)
    sp3 = x * dinv
    oa_ref[...] = _stack16(sp3[:, 0:32])
    ob_ref[...] = _stack16(sp3[:, 32:64])


def _tc4_body(ta_ref, tb_ref, spa_ref, spb_ref, dinv_ref, w3_ref, b3_ref,
              w4_ref, out_ref):
    dinv = dinv_ref[...]
    spa = spa_ref[...]
    spb = spb_ref[...]
    sp = jnp.concatenate([spa[0], spa[1], spb[0], spb[1]], axis=1)
    t = jnp.concatenate([ta_ref[...], tb_ref[...]], axis=1)
    u = dinv * (t + sp)
    x4 = jnp.maximum(jnp.dot(u, w3_ref[...],
                             preferred_element_type=jnp.float32) + b3_ref[...],
                     0.0)
    h4 = jnp.dot(x4, w4_ref[...], preferred_element_type=jnp.float32)
    out_ref[...] = _stack16(h4 * dinv)


def _tc5_body(t_ref, sp_ref, dinv_ref, b4_ref, w5_ref, out_ref):
    dinv = dinv_ref[...]
    sp = sp_ref[...]
    spc = jnp.concatenate([sp[0], sp[1]], axis=1)
    x5 = jnp.maximum(dinv * (t_ref[...] + spc) + b4_ref[...], 0.0)
    h5 = jnp.dot(x5, w5_ref[...], preferred_element_type=jnp.float32)
    out_ref[...] = h5 * dinv


def _tc6_body(t_ref, sp_ref, dinv_ref, b5_ref, out_ref):
    t = t_ref[...]
    dinv = dinv_ref[...]
    y = dinv * (t[0] + t[1] + sp_ref[...]) + b5_ref[...]
    out_ref[...] = y[:, 0:3]


def kernel(noisy_coords, atom_types, noisy_edge_index, atom_emb,
           W1, b1, W2, b2, W3, b3, W4, b4, W5, b5):
    n = noisy_coords.shape[0]
    e = noisy_edge_index.shape[1]
    np_rows = n + N_DUMP

    nchunk = -(-e // (32 * CHUNK))
    nchunk += (-nchunk) % IBLK
    ep = 32 * nchunk * CHUNK
    padlen = ep - e
    src = noisy_edge_index[0]
    dst = noisy_edge_index[1]
    pad_i = jnp.arange(padlen, dtype=jnp.int32)
    src_p = jnp.concatenate([src, (pad_i * 9973) % n])
    dst_p = jnp.concatenate([dst, n + (pad_i % 8)])
    srcp = src_p.reshape(32, nchunk // IBLK, IBLK, CHUNK)
    dstp = dst_p.reshape(32, nchunk // IBLK, IBLK, CHUNK)

    zeros16 = jnp.zeros((ZROWS, 16), jnp.float32)
    ones16 = jnp.ones((CHUNK, 16), jnp.float32)

    deg_pass = _sc_pass(16, nchunk, np_rows, with_gather=False)
    scat16 = _sc_pass(16, nchunk, np_rows, with_gather=True)
    scat32 = _sc_pass32(nchunk, np_rows)

    w1p = jnp.concatenate([W1, jnp.zeros((10, 32), jnp.float32)], axis=0)
    w5p = jnp.concatenate([W5, jnp.zeros((32, 13), jnp.float32)], axis=1)
    b1r = b1.reshape(1, -1)
    b2r = b2.reshape(1, -1)
    b3r = b3.reshape(1, -1)
    b4r = b4.reshape(1, -1)
    b5r = jnp.concatenate([b5, jnp.zeros((13,), jnp.float32)]).reshape(1, -1)
    e_pad = jnp.zeros((8, 128), jnp.float32).at[0:2, 0:3].set(atom_emb)
    atf = atom_types.astype(jnp.float32).reshape(n, 1)

    grid = (n // RB,)

    t_deg = deg_pass(ones16, srcp, dstp, zeros16)

    tc1 = _tc_call(
        _tc1_body,
        [_part_spec(16), _row_spec(3), _row_spec(1), _full_spec(8, 128)],
        [_row_spec(1), _row_spec(16)],
        (jax.ShapeDtypeStruct((n, 1), jnp.float32),
         jax.ShapeDtypeStruct((n, 16), jnp.float32)),
        grid)
    dinv, sp1 = tc1(t_deg, noisy_coords, atf, e_pad)

    t1 = scat16(sp1, srcp, dstp, zeros16)
    tc2 = _tc_call(
        _tc2_body,
        [_part_spec(16), _row_spec(16), _row_spec(1), _full_spec(16, 32),
         _full_spec(1, 32)],
        _part_spec(16),
        jax.ShapeDtypeStruct((2, n, 16), jnp.float32),
        grid)
    sp2 = tc2(t1, sp1, dinv, w1p, b1r)

    t2 = scat32(sp2, srcp, dstp, zeros16)
    tc3 = _tc_call(
        _tc3_body,
        [_row_spec(32), _part_spec(16), _row_spec(1), _full_spec(32, 64),
         _full_spec(1, 64)],
        [_part_spec(16), _part_spec(16)],
        (jax.ShapeDtypeStruct((2, n, 16), jnp.float32),
         jax.ShapeDtypeStruct((2, n, 16), jnp.float32)),
        grid)
    sp3a, sp3b = tc3(t2, sp2, dinv, W2, b2r)

    t3a = scat32(sp3a, srcp, dstp, zeros16)
    t3b = scat32(sp3b, srcp, dstp, zeros16)
    tc4 = _tc_call(
        _tc4_body,
        [_row_spec(32), _row_spec(32), _part_spec(16), _part_spec(16),
         _row_spec(1), _full_spec(64, 64), _full_spec(1, 64),
         _full_spec(64, 32)],
        _part_spec(16),
        jax.ShapeDtypeStruct((2, n, 16), jnp.float32),
        grid)
    sp4 = tc4(t3a, t3b, sp3a, sp3b, dinv, W3, b3r, W4)

    t4 = scat32(sp4, srcp, dstp, zeros16)
    tc5 = _tc_call(
        _tc5_body,
        [_row_spec(32), _part_spec(16), _row_spec(1), _full_spec(1, 32),
         _full_spec(32, 16)],
        _row_spec(16),
        jax.ShapeDtypeStruct((n, 16), jnp.float32),
        grid)
    sp5 = tc5(t4, sp4, dinv, b4r, w5p)

    t5 = scat16(sp5, srcp, dstp, zeros16)
    tc6 = _tc_call(
        _tc6_body,
        [_part_spec(16), _row_spec(16), _row_spec(1), _full_spec(1, 16)],
        _row_spec(3),
        jax.ShapeDtypeStruct((n, 3), jnp.float32),
        grid)
    return tc6(t5, sp5, dinv, b5r)


# speedup vs baseline: 1.7405x; 1.6756x over previous
import functools

import jax
import jax.numpy as jnp
from jax import lax
from jax.experimental import pallas as pl
from jax.experimental.pallas import tpu as pltpu
from jax.experimental.pallas import tpu_sc as plsc

CHUNK = 128
NBUF = 4
IBLK = 28
N_DUMP = 48
ZROWS = 1564
RB = 2000

_SC_PARAMS = pltpu.CompilerParams(use_tc_tiling_on_sc=False)


def _sc_pass(d, nchunk, np_rows, with_gather):
    mesh = plsc.VectorSubcoreMesh(core_axis_name="c", subcore_axis_name="s")
    rpt = np_rows // 16
    nblk = nchunk // IBLK
    nsb = IBLK // NBUF

    scratch = [
        pltpu.VMEM((IBLK, CHUNK), jnp.int32),
        pltpu.VMEM((IBLK, CHUNK), jnp.int32),
        pltpu.VMEM((NBUF, CHUNK, d), jnp.float32),
        pltpu.VMEM_SHARED((np_rows, d), jnp.float32),
        pltpu.SemaphoreType.DMA((NBUF,)),
        pltpu.SemaphoreType.DMA((NBUF,)),
    ]

    @functools.partial(
        pl.kernel, mesh=mesh,
        out_type=jax.ShapeDtypeStruct((2, np_rows, d), jnp.float32),
        scratch_types=scratch, compiler_params=_SC_PARAMS)
    def scat(s_hbm, srcp_hbm, dstp_hbm, zeros_hbm, out_hbm,
             src_v, dst_v, rows_v, acc, gsem, ssem):
        c = lax.axis_index("c")
        s = lax.axis_index("s")
        tid = c * 16 + s

        for z in range(rpt // ZROWS):
            pltpu.sync_copy(zeros_hbm, acc.at[pl.ds(s * rpt + z * ZROWS,
                                                    ZROWS)])
        if not with_gather:
            pltpu.sync_copy(s_hbm, rows_v.at[0])
        plsc.subcore_barrier()

        if with_gather:
            def blk(ob, carry):
                pltpu.sync_copy(srcp_hbm.at[tid, ob], src_v)
                pltpu.sync_copy(dstp_hbm.at[tid, ob], dst_v)

                def sb(i, carry2):
                    base = i * NBUF
                    first = (ob == 0) & (i == 0)
                    for k in range(NBUF):
                        @pl.when(jnp.logical_not(first))
                        def _():
                            pltpu.make_async_copy(
                                rows_v.at[k], acc.at[dst_v.at[base + k]],
                                ssem.at[k]).wait()
                        pltpu.async_copy(s_hbm.at[src_v.at[base + k]],
                                         rows_v.at[k], gsem.at[k])
                    for k in range(NBUF):
                        pltpu.make_async_copy(s_hbm.at[src_v.at[base + k]],
                                              rows_v.at[k],
                                              gsem.at[k]).wait()
                        pltpu.async_copy(rows_v.at[k],
                                         acc.at[dst_v.at[base + k]],
                                         ssem.at[k], add=True)
                    return carry2

                return lax.fori_loop(0, nsb, sb, carry)

            lax.fori_loop(0, nblk, blk, 0)
            for k in range(NBUF):
                pltpu.make_async_copy(rows_v.at[k], acc.at[dst_v.at[k]],
                                      ssem.at[k]).wait()
        else:
            def blk(ob, carry):
                pltpu.sync_copy(dstp_hbm.at[tid, ob], dst_v)

                def sb(i, carry2):
                    base = i * NBUF
                    first = (ob == 0) & (i == 0)
                    for k in range(NBUF):
                        @pl.when(jnp.logical_not(first))
                        def _():
                            pltpu.make_async_copy(
                                rows_v.at[0], acc.at[dst_v.at[base + k]],
                                ssem.at[k]).wait()
                        pltpu.async_copy(rows_v.at[0],
                                         acc.at[dst_v.at[base + k]],
                                         ssem.at[k], add=True)
                    return carry2

                return lax.fori_loop(0, nsb, sb, carry)

            lax.fori_loop(0, nblk, blk, 0)
            for k in range(NBUF):
                pltpu.make_async_copy(rows_v.at[0], acc.at[dst_v.at[k]],
                                      ssem.at[k]).wait()

        plsc.subcore_barrier()
        pltpu.sync_copy(acc.at[pl.ds(s * rpt, rpt)],
                        out_hbm.at[c, pl.ds(s * rpt, rpt)])

    return scat


PB = 1568
GRID = (8,)


def _pk():
    return pl.BlockSpec((PB, 128), lambda i: (i, 0))


def _pk2():
    return pl.BlockSpec((2, PB, 128), lambda i: (0, i, 0))


def _full_spec(r, ncol):
    return pl.BlockSpec((r, ncol), lambda i: (0, 0))


def _tc_call(body, in_specs, out_specs, out_shape):
    return pl.pallas_call(body, grid=GRID, in_specs=in_specs,
                          out_specs=out_specs, out_shape=out_shape)


def _mm(u, w_ref, b_ref):
    return jnp.dot(u, w_ref[...],
                   preferred_element_type=jnp.float32) + b_ref[...]


def _tc1_body(t_ref, cpack_ref, atp_ref, e0_ref, e1_ref, dinv_ref, sp1_ref):
    t = t_ref[...]
    dinv = lax.rsqrt(t[0] + t[1] + 1.0)
    emb = jnp.where(atp_ref[...] < 0.5, e0_ref[...], e1_ref[...])
    sp1_ref[...] = (cpack_ref[...] + emb) * dinv
    dinv_ref[...] = dinv


def _tc2_body(t_ref, sp_ref, dinv_ref, w_ref, b_ref, out_ref):
    t = t_ref[...]
    dinv = dinv_ref[...]
    u = dinv * (t[0] + t[1] + sp_ref[...])
    out_ref[...] = jnp.maximum(_mm(u, w_ref, b_ref), 0.0) * dinv


def _tc3_body(t_ref, sp_ref, dinv_ref, wa_ref, ba_ref, wb_ref, bb_ref,
              oa_ref, ob_ref):
    t = t_ref[...]
    dinv = dinv_ref[...]
    u = dinv * (t[0] + t[1] + sp_ref[...])
    oa_ref[...] = jnp.maximum(_mm(u, wa_ref, ba_ref), 0.0) * dinv
    ob_ref[...] = jnp.maximum(_mm(u, wb_ref, bb_ref), 0.0) * dinv


def _tc4_body(ta_ref, tb_ref, spa_ref, spb_ref, dinv_ref,
              w3aa_ref, w3ba_ref, ba_ref, w3ab_ref, w3bb_ref, bb_ref,
              w4a_ref, w4b_ref, out_ref):
    ta = ta_ref[...]
    tb = tb_ref[...]
    dinv = dinv_ref[...]
    ua = dinv * (ta[0] + ta[1] + spa_ref[...])
    ub = dinv * (tb[0] + tb[1] + spb_ref[...])
    x4a = jnp.maximum(
        jnp.dot(ua, w3aa_ref[...], preferred_element_type=jnp.float32)
        + jnp.dot(ub, w3ba_ref[...], preferred_element_type=jnp.float32)
        + ba_ref[...], 0.0)
    x4b = jnp.maximum(
        jnp.dot(ua, w3ab_ref[...], preferred_element_type=jnp.float32)
        + jnp.dot(ub, w3bb_ref[...], preferred_element_type=jnp.float32)
        + bb_ref[...], 0.0)
    h4 = (jnp.dot(x4a, w4a_ref[...], preferred_element_type=jnp.float32)
          + jnp.dot(x4b, w4b_ref[...], preferred_element_type=jnp.float32))
    out_ref[...] = h4 * dinv


def _tc5_body(t_ref, sp_ref, dinv_ref, b4_ref, w5_ref, z_ref, out_ref):
    t = t_ref[...]
    dinv = dinv_ref[...]
    x5 = jnp.maximum(dinv * (t[0] + t[1] + sp_ref[...]) + b4_ref[...], 0.0)
    out_ref[...] = _mm(x5, w5_ref, z_ref) * dinv


def _tc6_body(t_ref, sp_ref, dinv_ref, b5_ref, out_ref):
    t = t_ref[...]
    dinv = dinv_ref[...]
    out_ref[...] = dinv * (t[0] + t[1] + sp_ref[...]) + b5_ref[...]


def _bd4(w):
    return jnp.kron(jnp.eye(4, dtype=jnp.float32), w)


def _tile4(b):
    return jnp.tile(b, 4).reshape(1, 128)


def kernel(noisy_coords, atom_types, noisy_edge_index, atom_emb,
           W1, b1, W2, b2, W3, b3, W4, b4, W5, b5):
    n = noisy_coords.shape[0]
    e = noisy_edge_index.shape[1]
    np_rows = n + N_DUMP
    pk_rows = np_rows * 32 // 128

    nchunk = -(-e // (32 * CHUNK))
    nchunk += (-nchunk) % IBLK
    ep = 32 * nchunk * CHUNK
    padlen = ep - e
    src = noisy_edge_index[0]
    dst = noisy_edge_index[1]
    pad_i = jnp.arange(padlen, dtype=jnp.int32)
    src_p = jnp.concatenate([src, (pad_i * 9973) % n])
    dst_p = jnp.concatenate([dst, n + (pad_i % 8)])
    srcp = src_p.reshape(32, nchunk // IBLK, IBLK, CHUNK)
    dstp = dst_p.reshape(32, nchunk // IBLK, IBLK, CHUNK)

    zeros32 = jnp.zeros((ZROWS, 32), jnp.float32)
    ones32 = jnp.ones((CHUNK, 32), jnp.float32)

    deg_pass = _sc_pass(32, nchunk, np_rows, with_gather=False)
    scat32 = _sc_pass(32, nchunk, np_rows, with_gather=True)

    cpack = jnp.concatenate(
        [noisy_coords, jnp.zeros((n, 29), jnp.float32)], axis=1)
    cpack = jnp.concatenate(
        [cpack, jnp.zeros((N_DUMP, 32), jnp.float32)]).reshape(pk_rows, 128)
    atp = jnp.broadcast_to(
        jnp.concatenate([atom_types.astype(jnp.float32),
                         jnp.zeros((N_DUMP,), jnp.float32)])[:, None],
        (np_rows, 32)).reshape(pk_rows, 128)
    erow0 = jnp.concatenate([jnp.zeros((3,), jnp.float32), atom_emb[0],
                             jnp.zeros((26,), jnp.float32)])
    erow1 = jnp.concatenate([jnp.zeros((3,), jnp.float32), atom_emb[1],
                             jnp.zeros((26,), jnp.float32)])
    e0t = jnp.tile(erow0, 4).reshape(1, 128)
    e1t = jnp.tile(erow1, 4).reshape(1, 128)

    w1b = _bd4(jnp.concatenate([W1, jnp.zeros((26, 32), jnp.float32)],
                               axis=0))
    b1t = _tile4(b1)
    w2ab = _bd4(W2[:, 0:32])
    w2bb = _bd4(W2[:, 32:64])
    b2at = _tile4(b2[0:32])
    b2bt = _tile4(b2[32:64])
    w3aa = _bd4(W3[0:32, 0:32])
    w3ba = _bd4(W3[32:64, 0:32])
    w3ab = _bd4(W3[0:32, 32:64])
    w3bb = _bd4(W3[32:64, 32:64])
    b3at = _tile4(b3[0:32])
    b3bt = _tile4(b3[32:64])
    w4a = _bd4(W4[0:32, :])
    w4b = _bd4(W4[32:64, :])
    b4t = _tile4(b4)
    w5b = _bd4(jnp.concatenate([W5, jnp.zeros((32, 29), jnp.float32)],
                               axis=1))
    zt = jnp.zeros((1, 128), jnp.float32)
    b5t = _tile4(jnp.concatenate([b5, jnp.zeros((29,), jnp.float32)]))

    pkshape = jax.ShapeDtypeStruct((pk_rows, 128), jnp.float32)

    t_deg = deg_pass(ones32, srcp, dstp, zeros32).reshape(2, pk_rows, 128)

    tc1 = _tc_call(
        _tc1_body,
        [_pk2(), _pk(), _pk(), _full_spec(1, 128), _full_spec(1, 128)],
        [_pk(), _pk()], (pkshape, pkshape))
    dinv, sp1 = tc1(t_deg, cpack, atp, e0t, e1t)

    t1 = scat32(sp1.reshape(np_rows, 32), srcp, dstp,
                zeros32).reshape(2, pk_rows, 128)
    tc2 = _tc_call(
        _tc2_body,
        [_pk2(), _pk(), _pk(), _full_spec(128, 128), _full_spec(1, 128)],
        _pk(), pkshape)
    sp2 = tc2(t1, sp1, dinv, w1b, b1t)

    t2 = scat32(sp2.reshape(np_rows, 32), srcp, dstp,
                zeros32).reshape(2, pk_rows, 128)
    tc3 = _tc_call(
        _tc3_body,
        [_pk2(), _pk(), _pk(), _full_spec(128, 128), _full_spec(1, 128),
         _full_spec(128, 128), _full_spec(1, 128)],
        [_pk(), _pk()], (pkshape, pkshape))
    sp3a, sp3b = tc3(t2, sp2, dinv, w2ab, b2at, w2bb, b2bt)

    t3a = scat32(sp3a.reshape(np_rows, 32), srcp, dstp,
                 zeros32).reshape(2, pk_rows, 128)
    t3b = scat32(sp3b.reshape(np_rows, 32), srcp, dstp,
                 zeros32).reshape(2, pk_rows, 128)
    tc4 = _tc_call(
        _tc4_body,
        [_pk2(), _pk2(), _pk(), _pk(), _pk()]
        + [_full_spec(128, 128), _full_spec(128, 128), _full_spec(1, 128),
           _full_spec(128, 128), _full_spec(128, 128), _full_spec(1, 128),
           _full_spec(128, 128), _full_spec(128, 128)],
        _pk(), pkshape)
    sp4 = tc4(t3a, t3b, sp3a, sp3b, dinv, w3aa, w3ba, b3at, w3ab, w3bb,
              b3bt, w4a, w4b)

    t4 = scat32(sp4.reshape(np_rows, 32), srcp, dstp,
                zeros32).reshape(2, pk_rows, 128)
    tc5 = _tc_call(
        _tc5_body,
        [_pk2(), _pk(), _pk(), _full_spec(1, 128), _full_spec(128, 128),
         _full_spec(1, 128)],
        _pk(), pkshape)
    sp5 = tc5(t4, sp4, dinv, b4t, w5b, zt)

    t5 = scat32(sp5.reshape(np_rows, 32), srcp, dstp,
                zeros32).reshape(2, pk_rows, 128)
    tc6 = _tc_call(
        _tc6_body,
        [_pk2(), _pk(), _pk(), _full_spec(1, 128)],
        _pk(), pkshape)
    y = tc6(t5, sp5, dinv, b5t)
    return y.reshape(np_rows, 32)[:n, 0:3]
